# Initial kernel scaffold; baseline (speedup 1.0000x reference)
import jax
import jax.numpy as jnp
from jax.experimental import pallas as pl


def kernel(indices, embeddings):
    raise NotImplementedError("write your pallas kernel here")


# speedup vs baseline: 1.1093x; 1.1093x over previous
import functools

import jax
import jax.numpy as jnp
from jax import lax
from jax.experimental import pallas as pl
from jax.experimental.pallas import tpu as pltpu
from jax.experimental.pallas import tpu_sc as plsc


@functools.lru_cache(maxsize=None)
def _make_gather(total, d, n_workers, chunk):
    per_w = total // n_workers
    nchunk = per_w // chunk
    assert nchunk * chunk * n_workers == total
    mesh = plsc.VectorSubcoreMesh(core_axis_name="c", subcore_axis_name="s")

    @functools.partial(
        pl.kernel,
        mesh=mesh,
        out_type=jax.ShapeDtypeStruct((total, d), jnp.float32),
        compiler_params=pltpu.CompilerParams(use_tc_tiling_on_sc=False),
        scratch_types=[
            pltpu.VMEM((2, chunk), jnp.int32),
            pltpu.VMEM((2, chunk, d), jnp.float32),
            pltpu.SemaphoreType.DMA,
            pltpu.SemaphoreType.DMA,
            pltpu.SemaphoreType.DMA,
            pltpu.SemaphoreType.DMA,
            pltpu.SemaphoreType.DMA,
            pltpu.SemaphoreType.DMA,
        ],
    )
    def gather_kernel(idx_hbm, tab_hbm, out_hbm, idx_v, rows_v,
                      i_sem0, i_sem1, g_sem0, g_sem1, o_sem0, o_sem1):
        i_sems = (i_sem0, i_sem1)
        g_sems = (g_sem0, g_sem1)
        o_sems = (o_sem0, o_sem1)
        wid = lax.axis_index("s") * 2 + lax.axis_index("c")
        base = wid * per_w

        def idx_copy(g, b):
            return pltpu.make_async_copy(
                idx_hbm.at[pl.ds(base + g * chunk, chunk)],
                idx_v.at[b], i_sems[b])

        def gat_copy(b):
            return pltpu.make_async_copy(
                tab_hbm.at[idx_v.at[b]], rows_v.at[b], g_sems[b])

        def out_copy(g, b):
            return pltpu.make_async_copy(
                rows_v.at[b],
                out_hbm.at[pl.ds(base + g * chunk, chunk)],
                o_sems[b])

        idx_copy(0, 0).start()
        for g in range(nchunk):
            b = g & 1
            if g + 1 < nchunk:
                idx_copy(g + 1, 1 - b).start()
            idx_copy(g, b).wait()
            if g >= 2:
                out_copy(g - 2, b).wait()
            gat_copy(b).start()
            gat_copy(b).wait()
            out_copy(g, b).start()
        for g in range(max(0, nchunk - 2), nchunk):
            out_copy(g, g & 1).wait()

    return gather_kernel


def kernel(indices, embeddings):
    b, h = indices.shape
    v, d = embeddings.shape
    total = b * h
    info = plsc.get_sparse_core_info()
    n_workers = info.num_cores * info.num_subcores
    flat = indices.reshape(total).astype(jnp.int32)
    out = _make_gather(total, d, n_workers, 1280)(flat, embeddings)
    return out.reshape(b, h, d)


# speedup vs baseline: 1.8024x; 1.6248x over previous
import functools

import jax
import jax.numpy as jnp
from jax import lax
from jax.experimental import pallas as pl
from jax.experimental.pallas import tpu as pltpu
from jax.experimental.pallas import tpu_sc as plsc


@functools.lru_cache(maxsize=None)
def _make_gather(b, h, d, n_workers, chunk_b):
    rows_per_w = b // n_workers
    nchunk = rows_per_w // chunk_b
    assert nchunk * chunk_b * n_workers == b and nchunk % 2 == 0
    mesh = plsc.VectorSubcoreMesh(core_axis_name="c", subcore_axis_name="s")

    @functools.partial(
        pl.kernel,
        mesh=mesh,
        out_type=jax.ShapeDtypeStruct((b, h, d), jnp.float32),
        compiler_params=pltpu.CompilerParams(use_tc_tiling_on_sc=False),
        scratch_types=[
            pltpu.VMEM((2, chunk_b, h), jnp.int32),
            pltpu.VMEM((2, chunk_b, h, d), jnp.float32),
            pltpu.SemaphoreType.DMA,
            pltpu.SemaphoreType.DMA,
            pltpu.SemaphoreType.DMA,
            pltpu.SemaphoreType.DMA,
            pltpu.SemaphoreType.DMA,
            pltpu.SemaphoreType.DMA,
        ],
    )
    def gather_kernel(idx_hbm, tab_hbm, out_hbm, idx_v, rows_v,
                      i_sem0, i_sem1, g_sem0, g_sem1, o_sem0, o_sem1):
        i_sems = (i_sem0, i_sem1)
        g_sems = (g_sem0, g_sem1)
        o_sems = (o_sem0, o_sem1)
        wid = lax.axis_index("s") * 2 + lax.axis_index("c")
        base = wid * rows_per_w

        def idx_copy(g, s):
            return pltpu.make_async_copy(
                idx_hbm.at[pl.ds(base + g * chunk_b, chunk_b)],
                idx_v.at[s], i_sems[s])

        def gat_copy(s, j):
            return pltpu.make_async_copy(
                tab_hbm.at[idx_v.at[s, j]],
                rows_v.at[s, j], g_sems[s])

        def out_copy(g, s):
            return pltpu.make_async_copy(
                rows_v.at[s],
                out_hbm.at[pl.ds(base + g * chunk_b, chunk_b)],
                o_sems[s])

        idx_copy(0, 0).start()
        idx_copy(1, 1).start()

        def step(i, _):
            for s in (0, 1):
                g = 2 * i + s
                idx_copy(g, s).wait()

                @pl.when(g >= 2)
                def _():
                    out_copy(g - 2, s).wait()

                for j in range(chunk_b):
                    gat_copy(s, j).start()
                for j in range(chunk_b):
                    gat_copy(s, j).wait()
                out_copy(g, s).start()

                @pl.when(g + 2 < nchunk)
                def _():
                    idx_copy(g + 2, s).start()
            return _

        lax.fori_loop(0, nchunk // 2, step, None)
        out_copy(nchunk - 2, 0).wait()
        out_copy(nchunk - 1, 1).wait()

    return gather_kernel


def kernel(indices, embeddings):
    b, h = indices.shape
    v, d = embeddings.shape
    info = plsc.get_sparse_core_info()
    n_workers = info.num_cores * info.num_subcores
    return _make_gather(b, h, d, n_workers, 32)(indices.astype(jnp.int32),
                                                embeddings)
